# Initial kernel scaffold; baseline (speedup 1.0000x reference)
import jax
import jax.numpy as jnp
from jax.experimental import pallas as pl


def kernel(ind_2, p1, basis, W_pre1, b_pre1, W_pre2, b_pre2, W_pi, b_pi, W_ii1, W_ii2, W_po1, W_po2):
    raise NotImplementedError("write your pallas kernel here")


# speedup vs baseline: 3.6307x; 3.6307x over previous
import functools

import jax
import jax.numpy as jnp
from jax import lax
from jax.experimental import pallas as pl
from jax.experimental.pallas import tpu as pltpu
from jax.experimental.pallas import tpu_sc as plsc

_NC = 2
_NS = 16
_NW = _NC * _NS

_N_ATOMS = 10000
_N_PAIRS = 320000
_N_PAD = 10240
_ROWS_PER_TILE = _N_PAD // _NS
_PAIRS_PER_W = _N_PAIRS // _NW
_CHUNK = 1000
_N_CHUNKS = _PAIRS_PER_W // _CHUNK

_PAIR_BLOCK = 8000
_D = 16


def _f32(*shape):
    return jax.ShapeDtypeStruct(shape, jnp.float32)


def _pp_pre_body(p1, w1, b1, w2, b2, out):
    x = jnp.tanh(jnp.dot(p1[...], w1[...],
                         preferred_element_type=jnp.float32) + b1[...])
    out[...] = jnp.tanh(jnp.dot(x, w2[...],
                                preferred_element_type=jnp.float32) + b2[...])


def _pp_pre(p1, w1, b1, w2, b2):
    return pl.pallas_call(
        _pp_pre_body,
        out_shape=_f32(_N_ATOMS, _D),
    )(p1, w1, b1, w2, b2)


def _sc_gather_body(tbl, ind_i, ind_j, out_i, out_j,
                    idx_i, idx_j, rows_i, rows_j, sem_i, sem_j):
    wid = lax.axis_index("s") * _NC + lax.axis_index("c")
    base_w = wid * _PAIRS_PER_W
    for t in range(_N_CHUNKS):
        base = base_w + t * _CHUNK
        pltpu.sync_copy(ind_i.at[pl.ds(base, _CHUNK)], idx_i)
        pltpu.sync_copy(ind_j.at[pl.ds(base, _CHUNK)], idx_j)
        cp_i = pltpu.async_copy(tbl.at[idx_i], rows_i, sem_i)
        cp_j = pltpu.async_copy(tbl.at[idx_j], rows_j, sem_j)
        cp_i.wait()
        cp_j.wait()
        pltpu.sync_copy(rows_i, out_i.at[pl.ds(base, _CHUNK)])
        pltpu.sync_copy(rows_j, out_j.at[pl.ds(base, _CHUNK)])


@functools.cache
def _sc_gather():
    return functools.partial(
        pl.kernel,
        out_type=[_f32(_N_PAIRS, _D), _f32(_N_PAIRS, _D)],
        mesh=plsc.VectorSubcoreMesh(core_axis_name="c", subcore_axis_name="s",
                                    num_cores=_NC, num_subcores=_NS),
        scratch_types=[
            pltpu.VMEM((_CHUNK,), jnp.int32),
            pltpu.VMEM((_CHUNK,), jnp.int32),
            pltpu.VMEM((_CHUNK, _D), jnp.float32),
            pltpu.VMEM((_CHUNK, _D), jnp.float32),
            pltpu.SemaphoreType.DMA,
            pltpu.SemaphoreType.DMA,
        ],
        compiler_params=pltpu.CompilerParams(use_tc_tiling_on_sc=False),
    )(_sc_gather_body)


def _pair_body(pi, pj, basis, wa, wb, bpi, tmat, smat, wii1, wii2, out):
    u = (jnp.dot(pi[...], wa[...], preferred_element_type=jnp.float32)
         + jnp.dot(pj[...], wb[...], preferred_element_type=jnp.float32)
         + bpi[...])
    u = jnp.tanh(u)
    tiled = jnp.dot(basis[...], tmat[...], preferred_element_type=jnp.float32)
    w = jnp.dot(u * tiled, smat[...], preferred_element_type=jnp.float32)
    w = jnp.tanh(jnp.dot(w, wii1[...], preferred_element_type=jnp.float32))
    out[...] = jnp.tanh(jnp.dot(w, wii2[...],
                                preferred_element_type=jnp.float32))


def _pair_stage(pi, pj, basis, wa, wb, bpi, tmat, smat, wii1, wii2):
    nblk = _N_PAIRS // _PAIR_BLOCK
    blk = lambda r, c: pl.BlockSpec((r, c), lambda i: (i, 0))
    full = lambda r, c: pl.BlockSpec((r, c), lambda i: (0, 0))
    return pl.pallas_call(
        _pair_body,
        grid=(nblk,),
        in_specs=[
            blk(_PAIR_BLOCK, _D), blk(_PAIR_BLOCK, _D), blk(_PAIR_BLOCK, 16),
            full(16, 256), full(16, 256), full(1, 256),
            full(16, 256), full(256, 16),
            full(16, 16), full(16, 16),
        ],
        out_specs=blk(_PAIR_BLOCK, _D),
        out_shape=_f32(_N_PAIRS, _D),
    )(pi, pj, basis, wa, wb, bpi, tmat, smat, wii1, wii2)


def _sc_scatter_body(ipair, ind_i, zeros, out, idx_v, rows_v, agg, sem):
    cid = lax.axis_index("c")
    sid = lax.axis_index("s")
    wid = sid * _NC + cid
    r0 = sid * _ROWS_PER_TILE
    pltpu.sync_copy(zeros.at[pl.ds(r0, _ROWS_PER_TILE)],
                    agg.at[pl.ds(r0, _ROWS_PER_TILE)])
    plsc.subcore_barrier()
    base_w = wid * _PAIRS_PER_W
    for t in range(_N_CHUNKS):
        base = base_w + t * _CHUNK
        pltpu.sync_copy(ind_i.at[pl.ds(base, _CHUNK)], idx_v)
        pltpu.sync_copy(ipair.at[pl.ds(base, _CHUNK)], rows_v)
        pltpu.sync_copy(rows_v, agg.at[idx_v], add=True)
    plsc.subcore_barrier()
    pltpu.sync_copy(agg.at[pl.ds(r0, _ROWS_PER_TILE)],
                    out.at[cid, pl.ds(r0, _ROWS_PER_TILE)])


@functools.cache
def _sc_scatter():
    return functools.partial(
        pl.kernel,
        out_type=_f32(_NC, _N_PAD, _D),
        mesh=plsc.VectorSubcoreMesh(core_axis_name="c", subcore_axis_name="s",
                                    num_cores=_NC, num_subcores=_NS),
        scratch_types=[
            pltpu.VMEM((_CHUNK,), jnp.int32),
            pltpu.VMEM((_CHUNK, _D), jnp.float32),
            pltpu.VMEM_SHARED((_N_PAD, _D), jnp.float32),
            pltpu.SemaphoreType.DMA,
        ],
        compiler_params=pltpu.CompilerParams(use_tc_tiling_on_sc=False),
    )(_sc_scatter_body)


def _pp_post_body(a0, a1, w1, w2, out):
    agg = a0[...] + a1[...]
    x = jnp.tanh(jnp.dot(agg, w1[...], preferred_element_type=jnp.float32))
    out[...] = jnp.tanh(jnp.dot(x, w2[...], preferred_element_type=jnp.float32))


def _pp_post(a0, a1, w1, w2):
    return pl.pallas_call(
        _pp_post_body,
        out_shape=_f32(_N_ATOMS, _D),
    )(a0, a1, w1, w2)


def kernel(ind_2, p1, basis, W_pre1, b_pre1, W_pre2, b_pre2, W_pi, b_pi,
           W_ii1, W_ii2, W_po1, W_po2):
    ind_i = ind_2[:, 0]
    ind_j = ind_2[:, 1]

    p1_in = _pp_pre(p1, W_pre1, b_pre1.reshape(1, -1),
                    W_pre2, b_pre2.reshape(1, -1))

    prop_i, prop_j = _sc_gather()(p1_in, ind_i, ind_j)

    eye = jnp.eye(16, dtype=jnp.float32)
    tmat = jnp.tile(eye, (1, 16))
    smat = jnp.repeat(eye, 16, axis=0)
    i_pair = _pair_stage(prop_i, prop_j, basis,
                         W_pi[:16], W_pi[16:], b_pi.reshape(1, -1),
                         tmat, smat, W_ii1, W_ii2)

    zeros = jnp.zeros((_N_PAD, _D), jnp.float32)
    partials = _sc_scatter()(i_pair, ind_i, zeros)

    p1_new = _pp_post(partials[0, :_N_ATOMS], partials[1, :_N_ATOMS],
                      W_po1, W_po2)
    return (p1_new, i_pair)


# speedup vs baseline: 4.7786x; 1.3162x over previous
import functools

import jax
import jax.numpy as jnp
from jax import lax
from jax.experimental import pallas as pl
from jax.experimental.pallas import tpu as pltpu
from jax.experimental.pallas import tpu_sc as plsc

_NC = 2
_NS = 16
_NW = _NC * _NS

_N_ATOMS = 10000
_N_PAIRS = 320000
_N_PAD = 10240
_ROWS_PER_TILE = _N_PAD // _NS
_PAIRS_PER_W = _N_PAIRS // _NW
_CHUNK = 1000
_N_CHUNKS = _PAIRS_PER_W // _CHUNK

_PAIR_BLOCK = 8000
_D = 16


def _f32(*shape):
    return jax.ShapeDtypeStruct(shape, jnp.float32)


def _pp_pre_body(p1, w1, b1, w2, b2, out):
    x = jnp.tanh(jnp.dot(p1[...], w1[...],
                         preferred_element_type=jnp.float32) + b1[...])
    out[...] = jnp.tanh(jnp.dot(x, w2[...],
                                preferred_element_type=jnp.float32) + b2[...])


def _pp_pre(p1, w1, b1, w2, b2):
    return pl.pallas_call(
        _pp_pre_body,
        out_shape=_f32(_N_ATOMS, _D),
    )(p1, w1, b1, w2, b2)


def _sc_gather_body(tbl, ind_i, ind_j, out_i, out_j,
                    idx_i, idx_j, rows_i, rows_j, sem_i, sem_j):
    wid = lax.axis_index("s") * _NC + lax.axis_index("c")
    base_w = wid * _PAIRS_PER_W
    for t in range(_N_CHUNKS):
        base = base_w + t * _CHUNK
        pltpu.sync_copy(ind_i.at[pl.ds(base, _CHUNK)], idx_i)
        pltpu.sync_copy(ind_j.at[pl.ds(base, _CHUNK)], idx_j)
        cp_i = pltpu.async_copy(tbl.at[idx_i], rows_i, sem_i)
        cp_j = pltpu.async_copy(tbl.at[idx_j], rows_j, sem_j)
        cp_i.wait()
        cp_j.wait()
        pltpu.sync_copy(rows_i, out_i.at[pl.ds(base, _CHUNK)])
        pltpu.sync_copy(rows_j, out_j.at[pl.ds(base, _CHUNK)])


@functools.cache
def _sc_gather():
    return functools.partial(
        pl.kernel,
        out_type=[_f32(_N_PAIRS, _D), _f32(_N_PAIRS, _D)],
        mesh=plsc.VectorSubcoreMesh(core_axis_name="c", subcore_axis_name="s",
                                    num_cores=_NC, num_subcores=_NS),
        scratch_types=[
            pltpu.VMEM((_CHUNK,), jnp.int32),
            pltpu.VMEM((_CHUNK,), jnp.int32),
            pltpu.VMEM((_CHUNK, _D), jnp.float32),
            pltpu.VMEM((_CHUNK, _D), jnp.float32),
            pltpu.SemaphoreType.DMA,
            pltpu.SemaphoreType.DMA,
        ],
        compiler_params=pltpu.CompilerParams(use_tc_tiling_on_sc=False),
    )(_sc_gather_body)


def _pair_body(pi, pj, basis, wa, wb, bpi, tmat, smat, wii1, wii2, out):
    ys = []
    for k in range(8):
        sl = slice(16 * k, 16 * (k + 1))
        u = (jnp.dot(pi[:, sl], wa[...], preferred_element_type=jnp.float32)
             + jnp.dot(pj[:, sl], wb[...], preferred_element_type=jnp.float32)
             + bpi[...])
        u = jnp.tanh(u)
        tiled = jnp.dot(basis[:, sl], tmat[...],
                        preferred_element_type=jnp.float32)
        w = jnp.dot(u * tiled, smat[...], preferred_element_type=jnp.float32)
        w = jnp.tanh(jnp.dot(w, wii1[...], preferred_element_type=jnp.float32))
        ys.append(jnp.tanh(jnp.dot(w, wii2[...],
                                   preferred_element_type=jnp.float32)))
    out[...] = jnp.concatenate(ys, axis=1)


_PACK_ROWS = _N_PAIRS // 8
_PACK_BLOCK = _PAIR_BLOCK // 8


def _pair_stage(pi_p, pj_p, basis_p, wa, wb, bpi, tmat, smat, wii1, wii2):
    nblk = _PACK_ROWS // _PACK_BLOCK
    blk = lambda: pl.BlockSpec((_PACK_BLOCK, 128), lambda i: (i, 0))
    full = lambda r, c: pl.BlockSpec((r, c), lambda i: (0, 0))
    return pl.pallas_call(
        _pair_body,
        grid=(nblk,),
        in_specs=[
            blk(), blk(), blk(),
            full(16, 256), full(16, 256), full(1, 256),
            full(16, 256), full(256, 16),
            full(16, 16), full(16, 16),
        ],
        out_specs=blk(),
        out_shape=_f32(_PACK_ROWS, 128),
    )(pi_p, pj_p, basis_p, wa, wb, bpi, tmat, smat, wii1, wii2)


def _sc_scatter_body(ipair, ind_i, zeros, out, idx_v, rows_v, agg, sem):
    cid = lax.axis_index("c")
    sid = lax.axis_index("s")
    wid = sid * _NC + cid
    r0 = sid * _ROWS_PER_TILE
    pltpu.sync_copy(zeros.at[pl.ds(r0, _ROWS_PER_TILE)],
                    agg.at[pl.ds(r0, _ROWS_PER_TILE)])
    plsc.subcore_barrier()
    base_w = wid * _PAIRS_PER_W
    for t in range(_N_CHUNKS):
        base = base_w + t * _CHUNK
        pltpu.sync_copy(ind_i.at[pl.ds(base, _CHUNK)], idx_v)
        pltpu.sync_copy(ipair.at[pl.ds(base, _CHUNK)], rows_v)
        pltpu.sync_copy(rows_v, agg.at[idx_v], add=True)
    plsc.subcore_barrier()
    pltpu.sync_copy(agg.at[pl.ds(r0, _ROWS_PER_TILE)],
                    out.at[cid, pl.ds(r0, _ROWS_PER_TILE)])


@functools.cache
def _sc_scatter():
    return functools.partial(
        pl.kernel,
        out_type=_f32(_NC, _N_PAD, _D),
        mesh=plsc.VectorSubcoreMesh(core_axis_name="c", subcore_axis_name="s",
                                    num_cores=_NC, num_subcores=_NS),
        scratch_types=[
            pltpu.VMEM((_CHUNK,), jnp.int32),
            pltpu.VMEM((_CHUNK, _D), jnp.float32),
            pltpu.VMEM_SHARED((_N_PAD, _D), jnp.float32),
            pltpu.SemaphoreType.DMA,
        ],
        compiler_params=pltpu.CompilerParams(use_tc_tiling_on_sc=False),
    )(_sc_scatter_body)


def _pp_post_body(a0, a1, w1, w2, out):
    agg = a0[...] + a1[...]
    x = jnp.tanh(jnp.dot(agg, w1[...], preferred_element_type=jnp.float32))
    out[...] = jnp.tanh(jnp.dot(x, w2[...], preferred_element_type=jnp.float32))


def _pp_post(a0, a1, w1, w2):
    return pl.pallas_call(
        _pp_post_body,
        out_shape=_f32(_N_ATOMS, _D),
    )(a0, a1, w1, w2)


def kernel(ind_2, p1, basis, W_pre1, b_pre1, W_pre2, b_pre2, W_pi, b_pi,
           W_ii1, W_ii2, W_po1, W_po2):
    ind_i = ind_2[:, 0]
    ind_j = ind_2[:, 1]

    p1_in = _pp_pre(p1, W_pre1, b_pre1.reshape(1, -1),
                    W_pre2, b_pre2.reshape(1, -1))

    prop_i, prop_j = _sc_gather()(p1_in, ind_i, ind_j)

    eye = jnp.eye(16, dtype=jnp.float32)
    tmat = jnp.tile(eye, (1, 16))
    smat = jnp.repeat(eye, 16, axis=0)
    i_pair_p = _pair_stage(prop_i.reshape(_PACK_ROWS, 128),
                           prop_j.reshape(_PACK_ROWS, 128),
                           basis.reshape(_PACK_ROWS, 128),
                           W_pi[:16], W_pi[16:], b_pi.reshape(1, -1),
                           tmat, smat, W_ii1, W_ii2)
    i_pair = i_pair_p.reshape(_N_PAIRS, _D)

    zeros = jnp.zeros((_N_PAD, _D), jnp.float32)
    partials = _sc_scatter()(i_pair, ind_i, zeros)

    p1_new = _pp_post(partials[0, :_N_ATOMS], partials[1, :_N_ATOMS],
                      W_po1, W_po2)
    return (p1_new, i_pair)


# speedup vs baseline: 6.0432x; 1.2646x over previous
import functools

import jax
import jax.numpy as jnp
from jax import lax
from jax.experimental import pallas as pl
from jax.experimental.pallas import tpu as pltpu
from jax.experimental.pallas import tpu_sc as plsc

_NC = 2
_NS = 16
_NW = _NC * _NS

_N_ATOMS = 10000
_N_PAIRS = 320000
_N_PAD = 10240
_ROWS_PER_TILE = _N_PAD // _NS
_PAIRS_PER_W = _N_PAIRS // _NW
_CHUNK = 1000
_N_CHUNKS = _PAIRS_PER_W // _CHUNK

_PAIR_BLOCK = 8000
_D = 16


def _f32(*shape):
    return jax.ShapeDtypeStruct(shape, jnp.float32)


def _pp_pre_body(p1, w1, b1, w2, b2, out):
    x = jnp.tanh(jnp.dot(p1[...], w1[...],
                         preferred_element_type=jnp.float32) + b1[...])
    out[...] = jnp.tanh(jnp.dot(x, w2[...],
                                preferred_element_type=jnp.float32) + b2[...])


def _pp_pre(p1, w1, b1, w2, b2):
    return pl.pallas_call(
        _pp_pre_body,
        out_shape=_f32(_N_ATOMS, _D),
    )(p1, w1, b1, w2, b2)


def _sc_gather_body(tbl, ind_i, ind_j, out_i, out_j,
                    idx_i, idx_j, rows_i, rows_j, sem_i, sem_j):
    wid = lax.axis_index("s") * _NC + lax.axis_index("c")
    base_w = wid * _PAIRS_PER_W
    for t in range(_N_CHUNKS):
        base = base_w + t * _CHUNK
        pltpu.sync_copy(ind_i.at[pl.ds(base, _CHUNK)], idx_i)
        pltpu.sync_copy(ind_j.at[pl.ds(base, _CHUNK)], idx_j)
        cp_i = pltpu.async_copy(tbl.at[idx_i], rows_i, sem_i)
        cp_j = pltpu.async_copy(tbl.at[idx_j], rows_j, sem_j)
        cp_i.wait()
        cp_j.wait()
        pltpu.sync_copy(rows_i, out_i.at[pl.ds(base, _CHUNK)])
        pltpu.sync_copy(rows_j, out_j.at[pl.ds(base, _CHUNK)])


@functools.cache
def _sc_gather():
    return functools.partial(
        pl.kernel,
        out_type=[_f32(_N_PAIRS, _D), _f32(_N_PAIRS, _D)],
        mesh=plsc.VectorSubcoreMesh(core_axis_name="c", subcore_axis_name="s",
                                    num_cores=_NC, num_subcores=_NS),
        scratch_types=[
            pltpu.VMEM((_CHUNK,), jnp.int32),
            pltpu.VMEM((_CHUNK,), jnp.int32),
            pltpu.VMEM((_CHUNK, _D), jnp.float32),
            pltpu.VMEM((_CHUNK, _D), jnp.float32),
            pltpu.SemaphoreType.DMA,
            pltpu.SemaphoreType.DMA,
        ],
        compiler_params=pltpu.CompilerParams(use_tc_tiling_on_sc=False),
    )(_sc_gather_body)


_PACK_ROWS = _N_PAIRS // 8
_PACK_BLOCK = 2000
_GRP = _PACK_BLOCK


def _pair_body(pi, pj, bs_t, wa, wb, bpi, tmat, eye, smat, wii1, wii2,
               out_pk, out_t):
    for k in range(8):
        sl = slice(16 * k, 16 * (k + 1))
        u = (jnp.dot(pi[:, sl], wa[...], preferred_element_type=jnp.float32)
             + jnp.dot(pj[:, sl], wb[...], preferred_element_type=jnp.float32)
             + bpi[...])
        u = jnp.tanh(u)
        bs_k = bs_t[:, _GRP * k:_GRP * (k + 1)]
        tiled = jax.lax.dot_general(bs_k, tmat[...], (((0,), (0,)), ((), ())),
                                    preferred_element_type=jnp.float32)
        w = jnp.dot(u * tiled, smat[...], preferred_element_type=jnp.float32)
        w = jnp.tanh(jnp.dot(w, wii1[...], preferred_element_type=jnp.float32))
        y = jnp.tanh(jnp.dot(w, wii2[...], preferred_element_type=jnp.float32))
        out_pk[:, sl] = y
        y_t = jax.lax.dot_general(eye[...], y, (((1,), (1,)), ((), ())),
                                  preferred_element_type=jnp.float32)
        out_t[:, _GRP * k:_GRP * (k + 1)] = y_t


def _pair_stage(pi_p, pj_p, basis_t, wa, wb, bpi, tmat, eye, smat, wii1, wii2):
    nblk = _PACK_ROWS // _PACK_BLOCK
    blk = lambda: pl.BlockSpec((_PACK_BLOCK, 128), lambda i: (i, 0))
    tblk = lambda: pl.BlockSpec((16, 8 * _GRP), lambda i: (0, i))
    full = lambda r, c: pl.BlockSpec((r, c), lambda i: (0, 0))
    return pl.pallas_call(
        _pair_body,
        grid=(nblk,),
        in_specs=[
            blk(), blk(), tblk(),
            full(16, 256), full(16, 256), full(1, 256),
            full(16, 256), full(16, 16), full(256, 16),
            full(16, 16), full(16, 16),
        ],
        out_specs=[blk(), tblk()],
        out_shape=[_f32(_PACK_ROWS, 128), _f32(16, _N_PAIRS)],
    )(pi_p, pj_p, basis_t, wa, wb, bpi, tmat, eye, smat, wii1, wii2)


def _sc_scatter_body(ipair, ind_i, zeros, out, idx_v, rows_v, agg, sem):
    cid = lax.axis_index("c")
    sid = lax.axis_index("s")
    wid = sid * _NC + cid
    r0 = sid * _ROWS_PER_TILE
    pltpu.sync_copy(zeros.at[pl.ds(r0, _ROWS_PER_TILE)],
                    agg.at[pl.ds(r0, _ROWS_PER_TILE)])
    plsc.subcore_barrier()
    base_w = wid * _PAIRS_PER_W
    for t in range(_N_CHUNKS):
        base = base_w + t * _CHUNK
        pltpu.sync_copy(ind_i.at[pl.ds(base, _CHUNK)], idx_v)
        pltpu.sync_copy(ipair.at[pl.ds(base, _CHUNK)], rows_v)
        pltpu.sync_copy(rows_v, agg.at[idx_v], add=True)
    plsc.subcore_barrier()
    pltpu.sync_copy(agg.at[pl.ds(r0, _ROWS_PER_TILE)],
                    out.at[cid, pl.ds(r0, _ROWS_PER_TILE)])


@functools.cache
def _sc_scatter():
    return functools.partial(
        pl.kernel,
        out_type=_f32(_NC, _N_PAD, _D),
        mesh=plsc.VectorSubcoreMesh(core_axis_name="c", subcore_axis_name="s",
                                    num_cores=_NC, num_subcores=_NS),
        scratch_types=[
            pltpu.VMEM((_CHUNK,), jnp.int32),
            pltpu.VMEM((_CHUNK, _D), jnp.float32),
            pltpu.VMEM_SHARED((_N_PAD, _D), jnp.float32),
            pltpu.SemaphoreType.DMA,
        ],
        compiler_params=pltpu.CompilerParams(use_tc_tiling_on_sc=False),
    )(_sc_scatter_body)


def _pp_post_body(a0, a1, w1, w2, out):
    agg = a0[...] + a1[...]
    x = jnp.tanh(jnp.dot(agg, w1[...], preferred_element_type=jnp.float32))
    out[...] = jnp.tanh(jnp.dot(x, w2[...], preferred_element_type=jnp.float32))


def _pp_post(a0, a1, w1, w2):
    return pl.pallas_call(
        _pp_post_body,
        out_shape=_f32(_N_ATOMS, _D),
    )(a0, a1, w1, w2)


def _perm(x):
    nblk = _PACK_ROWS // _PACK_BLOCK
    return x.reshape(nblk, 8, _GRP).transpose(0, 2, 1).reshape(-1)


def kernel(ind_2, p1, basis, W_pre1, b_pre1, W_pre2, b_pre2, W_pi, b_pi,
           W_ii1, W_ii2, W_po1, W_po2):
    ind_i = _perm(ind_2[:, 0])
    ind_j = _perm(ind_2[:, 1])

    p1_in = _pp_pre(p1, W_pre1, b_pre1.reshape(1, -1),
                    W_pre2, b_pre2.reshape(1, -1))

    prop_i, prop_j = _sc_gather()(p1_in, ind_i, ind_j)

    eye = jnp.eye(16, dtype=jnp.float32)
    tmat = jnp.tile(eye, (1, 16))
    smat = jnp.repeat(eye, 16, axis=0)
    i_pair_pk, i_pair_t = _pair_stage(
        prop_i.reshape(_PACK_ROWS, 128), prop_j.reshape(_PACK_ROWS, 128),
        basis.T,
        W_pi[:16], W_pi[16:], b_pi.reshape(1, -1),
        tmat, eye, smat, W_ii1, W_ii2)

    zeros = jnp.zeros((_N_PAD, _D), jnp.float32)
    partials = _sc_scatter()(i_pair_pk.reshape(_N_PAIRS, _D), ind_i, zeros)

    p1_new = _pp_post(partials[0, :_N_ATOMS], partials[1, :_N_ATOMS],
                      W_po1, W_po2)
    return (p1_new, i_pair_t.T)


# speedup vs baseline: 6.0729x; 1.0049x over previous
import functools

import jax
import jax.numpy as jnp
from jax import lax
from jax.experimental import pallas as pl
from jax.experimental.pallas import tpu as pltpu
from jax.experimental.pallas import tpu_sc as plsc

_NC = 2
_NS = 16
_NW = _NC * _NS

_N_ATOMS = 10000
_N_PAIRS = 320000
_N_PAD = 10240
_ROWS_PER_TILE = _N_PAD // _NS
_PAIRS_PER_W = _N_PAIRS // _NW
_CHUNK = 1000
_N_CHUNKS = _PAIRS_PER_W // _CHUNK

_PAIR_BLOCK = 8000
_D = 16


def _f32(*shape):
    return jax.ShapeDtypeStruct(shape, jnp.float32)


def _pp_pre_body(p1, w1, b1, w2, b2, out):
    x = jnp.tanh(jnp.dot(p1[...], w1[...],
                         preferred_element_type=jnp.float32) + b1[...])
    out[...] = jnp.tanh(jnp.dot(x, w2[...],
                                preferred_element_type=jnp.float32) + b2[...])


def _pp_pre(p1, w1, b1, w2, b2):
    return pl.pallas_call(
        _pp_pre_body,
        out_shape=_f32(_N_ATOMS, _D),
    )(p1, w1, b1, w2, b2)


def _sc_gather_body(tbl, ind_i, ind_j, out_i, out_j,
                    idx_i, idx_j, rows_i, rows_j, sem_i, sem_j):
    wid = lax.axis_index("s") * _NC + lax.axis_index("c")
    base_w = wid * _PAIRS_PER_W
    for t in range(_N_CHUNKS):
        base = base_w + t * _CHUNK
        pltpu.sync_copy(ind_i.at[pl.ds(base, _CHUNK)], idx_i)
        pltpu.sync_copy(ind_j.at[pl.ds(base, _CHUNK)], idx_j)
        cp_i = pltpu.async_copy(tbl.at[idx_i], rows_i, sem_i)
        cp_j = pltpu.async_copy(tbl.at[idx_j], rows_j, sem_j)
        cp_i.wait()
        cp_j.wait()
        pltpu.sync_copy(rows_i, out_i.at[pl.ds(base, _CHUNK)])
        pltpu.sync_copy(rows_j, out_j.at[pl.ds(base, _CHUNK)])


@functools.cache
def _sc_gather():
    return functools.partial(
        pl.kernel,
        out_type=[_f32(_N_PAIRS, _D), _f32(_N_PAIRS, _D)],
        mesh=plsc.VectorSubcoreMesh(core_axis_name="c", subcore_axis_name="s",
                                    num_cores=_NC, num_subcores=_NS),
        scratch_types=[
            pltpu.VMEM((_CHUNK,), jnp.int32),
            pltpu.VMEM((_CHUNK,), jnp.int32),
            pltpu.VMEM((_CHUNK, _D), jnp.float32),
            pltpu.VMEM((_CHUNK, _D), jnp.float32),
            pltpu.SemaphoreType.DMA,
            pltpu.SemaphoreType.DMA,
        ],
        compiler_params=pltpu.CompilerParams(use_tc_tiling_on_sc=False),
    )(_sc_gather_body)


_PACK_ROWS = _N_PAIRS // 8
_PACK_BLOCK = 2000
_GRP = _PACK_BLOCK


def _pair_body(pi, pj, bs_t, wbig, bpi8, tmat, sbig, wii1b, wii2b,
               out_pk, out_t):
    cat = jnp.concatenate([pi[...], pj[...]], axis=1)
    u = jnp.dot(cat, wbig[...], preferred_element_type=jnp.float32) + bpi8[...]
    u = jnp.tanh(u)
    b_all = bs_t[...].T
    tiled = jnp.concatenate(
        [jnp.dot(b_all[_GRP * k:_GRP * (k + 1)], tmat[...],
                 preferred_element_type=jnp.float32) for k in range(8)],
        axis=1)
    w = jnp.dot(u * tiled, sbig[...], preferred_element_type=jnp.float32)
    w = jnp.tanh(jnp.dot(w, wii1b[...], preferred_element_type=jnp.float32))
    y = jnp.tanh(jnp.dot(w, wii2b[...], preferred_element_type=jnp.float32))
    out_pk[...] = y
    y_all = jnp.concatenate(
        [y[:, 16 * k:16 * (k + 1)] for k in range(8)], axis=0)
    out_t[...] = y_all.T


def _pair_stage(pi_p, pj_p, basis_t, wbig, bpi8, tmat, sbig, wii1b, wii2b):
    nblk = _PACK_ROWS // _PACK_BLOCK
    blk = lambda: pl.BlockSpec((_PACK_BLOCK, 128), lambda i: (i, 0))
    tblk = lambda: pl.BlockSpec((16, 8 * _GRP), lambda i: (0, i))
    full = lambda r, c: pl.BlockSpec((r, c), lambda i: (0, 0))
    return pl.pallas_call(
        _pair_body,
        grid=(nblk,),
        in_specs=[
            blk(), blk(), tblk(),
            full(256, 2048), full(1, 2048),
            full(16, 256), full(2048, 128),
            full(128, 128), full(128, 128),
        ],
        out_specs=[blk(), tblk()],
        out_shape=[_f32(_PACK_ROWS, 128), _f32(16, _N_PAIRS)],
        compiler_params=pltpu.CompilerParams(fuse_transposed_lhs_in_matmul=True),
    )(pi_p, pj_p, basis_t, wbig, bpi8, tmat, sbig, wii1b, wii2b)


def _sc_scatter_body(ipair, ind_i, zeros, out, idx_v, rows_v, agg, sem):
    cid = lax.axis_index("c")
    sid = lax.axis_index("s")
    wid = sid * _NC + cid
    r0 = sid * _ROWS_PER_TILE
    pltpu.sync_copy(zeros.at[pl.ds(r0, _ROWS_PER_TILE)],
                    agg.at[pl.ds(r0, _ROWS_PER_TILE)])
    plsc.subcore_barrier()
    base_w = wid * _PAIRS_PER_W
    for t in range(_N_CHUNKS):
        base = base_w + t * _CHUNK
        pltpu.sync_copy(ind_i.at[pl.ds(base, _CHUNK)], idx_v)
        pltpu.sync_copy(ipair.at[pl.ds(base, _CHUNK)], rows_v)
        pltpu.sync_copy(rows_v, agg.at[idx_v], add=True)
    plsc.subcore_barrier()
    pltpu.sync_copy(agg.at[pl.ds(r0, _ROWS_PER_TILE)],
                    out.at[cid, pl.ds(r0, _ROWS_PER_TILE)])


@functools.cache
def _sc_scatter():
    return functools.partial(
        pl.kernel,
        out_type=_f32(_NC, _N_PAD, _D),
        mesh=plsc.VectorSubcoreMesh(core_axis_name="c", subcore_axis_name="s",
                                    num_cores=_NC, num_subcores=_NS),
        scratch_types=[
            pltpu.VMEM((_CHUNK,), jnp.int32),
            pltpu.VMEM((_CHUNK, _D), jnp.float32),
            pltpu.VMEM_SHARED((_N_PAD, _D), jnp.float32),
            pltpu.SemaphoreType.DMA,
        ],
        compiler_params=pltpu.CompilerParams(use_tc_tiling_on_sc=False),
    )(_sc_scatter_body)


def _pp_post_body(a0, a1, w1, w2, out):
    agg = a0[...] + a1[...]
    x = jnp.tanh(jnp.dot(agg, w1[...], preferred_element_type=jnp.float32))
    out[...] = jnp.tanh(jnp.dot(x, w2[...], preferred_element_type=jnp.float32))


def _pp_post(a0, a1, w1, w2):
    return pl.pallas_call(
        _pp_post_body,
        out_shape=_f32(_N_ATOMS, _D),
    )(a0, a1, w1, w2)


def _perm(x):
    nblk = _PACK_ROWS // _PACK_BLOCK
    return x.reshape(nblk, 8, _GRP).transpose(0, 2, 1).reshape(-1)


def kernel(ind_2, p1, basis, W_pre1, b_pre1, W_pre2, b_pre2, W_pi, b_pi,
           W_ii1, W_ii2, W_po1, W_po2):
    ind_i = _perm(ind_2[:, 0])
    ind_j = _perm(ind_2[:, 1])

    p1_in = _pp_pre(p1, W_pre1, b_pre1.reshape(1, -1),
                    W_pre2, b_pre2.reshape(1, -1))

    prop_i, prop_j = _sc_gather()(p1_in, ind_i, ind_j)

    eye16 = jnp.eye(16, dtype=jnp.float32)
    eye8 = jnp.eye(8, dtype=jnp.float32)
    tmat = jnp.tile(eye16, (1, 16))
    smat = jnp.repeat(eye16, 16, axis=0)
    wbig = jnp.concatenate([jnp.kron(eye8, W_pi[:16]),
                            jnp.kron(eye8, W_pi[16:])], axis=0)
    bpi8 = jnp.tile(b_pi, 8).reshape(1, -1)
    sbig = jnp.kron(eye8, smat)
    wii1b = jnp.kron(eye8, W_ii1)
    wii2b = jnp.kron(eye8, W_ii2)
    i_pair_pk, i_pair_t = _pair_stage(
        prop_i.reshape(_PACK_ROWS, 128), prop_j.reshape(_PACK_ROWS, 128),
        basis.T, wbig, bpi8, tmat, sbig, wii1b, wii2b)

    zeros = jnp.zeros((_N_PAD, _D), jnp.float32)
    partials = _sc_scatter()(i_pair_pk.reshape(_N_PAIRS, _D), ind_i, zeros)

    p1_new = _pp_post(partials[0, :_N_ATOMS], partials[1, :_N_ATOMS],
                      W_po1, W_po2)
    return (p1_new, i_pair_t.T)


# speedup vs baseline: 7.2584x; 1.1952x over previous
import functools

import jax
import jax.numpy as jnp
from jax import lax
from jax.experimental import pallas as pl
from jax.experimental.pallas import tpu as pltpu
from jax.experimental.pallas import tpu_sc as plsc

_NC = 2
_NS = 16
_NW = _NC * _NS

_N_ATOMS = 10000
_N_PAIRS = 320000
_N_PAD = 10240
_ROWS_PER_TILE = _N_PAD // _NS
_PAIRS_PER_W = _N_PAIRS // _NW
_CHUNK = 1000
_N_CHUNKS = _PAIRS_PER_W // _CHUNK

_PAIR_BLOCK = 8000
_D = 16


def _f32(*shape):
    return jax.ShapeDtypeStruct(shape, jnp.float32)


def _pp_pre_body(p1, w1, b1, w2, b2, out):
    x = jnp.tanh(jnp.dot(p1[...], w1[...],
                         preferred_element_type=jnp.float32) + b1[...])
    out[...] = jnp.tanh(jnp.dot(x, w2[...],
                                preferred_element_type=jnp.float32) + b2[...])


def _pp_pre(p1, w1, b1, w2, b2):
    return pl.pallas_call(
        _pp_pre_body,
        out_shape=_f32(_N_ATOMS, _D),
    )(p1, w1, b1, w2, b2)


def _sc_gather_body(tbl, ind_i, ind_j, out_i, out_j,
                    idx_i, idx_j, rows_i, rows_j, sem_i, sem_j):
    wid = lax.axis_index("s") * _NC + lax.axis_index("c")
    base_w = wid * _PAIRS_PER_W
    for t in range(_N_CHUNKS):
        base = base_w + t * _CHUNK
        pltpu.sync_copy(ind_i.at[pl.ds(base, _CHUNK)], idx_i)
        pltpu.sync_copy(ind_j.at[pl.ds(base, _CHUNK)], idx_j)
        cp_i = pltpu.async_copy(tbl.at[idx_i], rows_i, sem_i)
        cp_j = pltpu.async_copy(tbl.at[idx_j], rows_j, sem_j)
        cp_i.wait()
        cp_j.wait()
        pltpu.sync_copy(rows_i, out_i.at[pl.ds(base, _CHUNK)])
        pltpu.sync_copy(rows_j, out_j.at[pl.ds(base, _CHUNK)])


@functools.cache
def _sc_gather():
    return functools.partial(
        pl.kernel,
        out_type=[_f32(_N_PAIRS, _D), _f32(_N_PAIRS, _D)],
        mesh=plsc.VectorSubcoreMesh(core_axis_name="c", subcore_axis_name="s",
                                    num_cores=_NC, num_subcores=_NS),
        scratch_types=[
            pltpu.VMEM((_CHUNK,), jnp.int32),
            pltpu.VMEM((_CHUNK,), jnp.int32),
            pltpu.VMEM((_CHUNK, _D), jnp.float32),
            pltpu.VMEM((_CHUNK, _D), jnp.float32),
            pltpu.SemaphoreType.DMA,
            pltpu.SemaphoreType.DMA,
        ],
        compiler_params=pltpu.CompilerParams(use_tc_tiling_on_sc=False),
    )(_sc_gather_body)


_PACK_ROWS = _N_PAIRS // 8
_PACK_BLOCK = 800
_GRP = _PACK_BLOCK


def _pair_body(pi, pj, bs_t, wbig, bpi8, tmat, sbig, wii1b, wii2b,
               wii2bt, out_pk, out_t):
    cat = jnp.concatenate([pi[...], pj[...]], axis=1)
    u = jnp.dot(cat, wbig[...], preferred_element_type=jnp.float32) + bpi8[...]
    u = jnp.tanh(u)
    b_all = bs_t[...].T
    tiled = jnp.concatenate(
        [jnp.dot(b_all[_GRP * k:_GRP * (k + 1)], tmat[...],
                 preferred_element_type=jnp.float32) for k in range(8)],
        axis=1)
    w = jnp.dot(u * tiled, sbig[...], preferred_element_type=jnp.float32)
    w = jnp.tanh(jnp.dot(w, wii1b[...], preferred_element_type=jnp.float32))
    y = jnp.tanh(jnp.dot(w, wii2b[...], preferred_element_type=jnp.float32))
    out_pk[...] = y
    yt = jnp.tanh(jnp.dot(wii2bt[...], w.T,
                          preferred_element_type=jnp.float32))
    out_t[...] = jnp.concatenate(
        [yt[16 * k:16 * (k + 1), :] for k in range(8)], axis=1)


def _pair_stage(pi_p, pj_p, basis_t, wbig, bpi8, tmat, sbig, wii1b, wii2b,
                wii2bt):
    nblk = _PACK_ROWS // _PACK_BLOCK
    blk = lambda: pl.BlockSpec((_PACK_BLOCK, 128), lambda i: (i, 0))
    tblk = lambda: pl.BlockSpec((16, 8 * _GRP), lambda i: (0, i))
    full = lambda r, c: pl.BlockSpec((r, c), lambda i: (0, 0))
    return pl.pallas_call(
        _pair_body,
        grid=(nblk,),
        in_specs=[
            blk(), blk(), tblk(),
            full(256, 2048), full(1, 2048),
            full(16, 256), full(2048, 128),
            full(128, 128), full(128, 128), full(128, 128),
        ],
        out_specs=[blk(), tblk()],
        out_shape=[_f32(_PACK_ROWS, 128), _f32(16, _N_PAIRS)],
        compiler_params=pltpu.CompilerParams(fuse_transposed_lhs_in_matmul=True),
    )(pi_p, pj_p, basis_t, wbig, bpi8, tmat, sbig, wii1b, wii2b, wii2bt)


def _sc_scatter_body(ipair, ind_i, zeros, out, idx_v, rows_v, agg, sem):
    cid = lax.axis_index("c")
    sid = lax.axis_index("s")
    wid = sid * _NC + cid
    r0 = sid * _ROWS_PER_TILE
    pltpu.sync_copy(zeros.at[pl.ds(r0, _ROWS_PER_TILE)],
                    agg.at[pl.ds(r0, _ROWS_PER_TILE)])
    plsc.subcore_barrier()
    base_w = wid * _PAIRS_PER_W
    for t in range(_N_CHUNKS):
        base = base_w + t * _CHUNK
        pltpu.sync_copy(ind_i.at[pl.ds(base, _CHUNK)], idx_v)
        pltpu.sync_copy(ipair.at[pl.ds(base, _CHUNK)], rows_v)
        pltpu.sync_copy(rows_v, agg.at[idx_v], add=True)
    plsc.subcore_barrier()
    pltpu.sync_copy(agg.at[pl.ds(r0, _ROWS_PER_TILE)],
                    out.at[cid, pl.ds(r0, _ROWS_PER_TILE)])


@functools.cache
def _sc_scatter():
    return functools.partial(
        pl.kernel,
        out_type=_f32(_NC, _N_PAD, _D),
        mesh=plsc.VectorSubcoreMesh(core_axis_name="c", subcore_axis_name="s",
                                    num_cores=_NC, num_subcores=_NS),
        scratch_types=[
            pltpu.VMEM((_CHUNK,), jnp.int32),
            pltpu.VMEM((_CHUNK, _D), jnp.float32),
            pltpu.VMEM_SHARED((_N_PAD, _D), jnp.float32),
            pltpu.SemaphoreType.DMA,
        ],
        compiler_params=pltpu.CompilerParams(use_tc_tiling_on_sc=False),
    )(_sc_scatter_body)


def _pp_post_body(a0, a1, w1, w2, out):
    agg = a0[...] + a1[...]
    x = jnp.tanh(jnp.dot(agg, w1[...], preferred_element_type=jnp.float32))
    out[...] = jnp.tanh(jnp.dot(x, w2[...], preferred_element_type=jnp.float32))


def _pp_post(a0, a1, w1, w2):
    return pl.pallas_call(
        _pp_post_body,
        out_shape=_f32(_N_ATOMS, _D),
    )(a0, a1, w1, w2)


def _perm(x):
    nblk = _PACK_ROWS // _PACK_BLOCK
    return x.reshape(nblk, 8, _GRP).transpose(0, 2, 1).reshape(-1)


def kernel(ind_2, p1, basis, W_pre1, b_pre1, W_pre2, b_pre2, W_pi, b_pi,
           W_ii1, W_ii2, W_po1, W_po2):
    ind_i = _perm(ind_2[:, 0])
    ind_j = _perm(ind_2[:, 1])

    p1_in = _pp_pre(p1, W_pre1, b_pre1.reshape(1, -1),
                    W_pre2, b_pre2.reshape(1, -1))

    prop_i, prop_j = _sc_gather()(p1_in, ind_i, ind_j)

    eye16 = jnp.eye(16, dtype=jnp.float32)
    eye8 = jnp.eye(8, dtype=jnp.float32)
    tmat = jnp.tile(eye16, (1, 16))
    smat = jnp.repeat(eye16, 16, axis=0)
    wbig = jnp.concatenate([jnp.kron(eye8, W_pi[:16]),
                            jnp.kron(eye8, W_pi[16:])], axis=0)
    bpi8 = jnp.tile(b_pi, 8).reshape(1, -1)
    sbig = jnp.kron(eye8, smat)
    wii1b = jnp.kron(eye8, W_ii1)
    wii2b = jnp.kron(eye8, W_ii2)
    wii2bt = jnp.kron(eye8, W_ii2.T)
    i_pair_pk, i_pair_t = _pair_stage(
        prop_i.reshape(_PACK_ROWS, 128), prop_j.reshape(_PACK_ROWS, 128),
        basis.T, wbig, bpi8, tmat, sbig, wii1b, wii2b, wii2bt)

    zeros = jnp.zeros((_N_PAD, _D), jnp.float32)
    partials = _sc_scatter()(i_pair_pk.reshape(_N_PAIRS, _D), ind_i, zeros)

    p1_new = _pp_post(partials[0, :_N_ATOMS], partials[1, :_N_ATOMS],
                      W_po1, W_po2)
    return (p1_new, i_pair_t.T)


# speedup vs baseline: 7.6283x; 1.0510x over previous
import functools

import jax
import jax.numpy as jnp
from jax import lax
from jax.experimental import pallas as pl
from jax.experimental.pallas import tpu as pltpu
from jax.experimental.pallas import tpu_sc as plsc

_NC = 2
_NS = 16
_NW = _NC * _NS

_N_ATOMS = 10000
_N_PAIRS = 320000
_N_PAD = 10240
_ROWS_PER_TILE = _N_PAD // _NS
_PAIRS_PER_W = _N_PAIRS // _NW
_CHUNK = 1000
_N_CHUNKS = _PAIRS_PER_W // _CHUNK

_PAIR_BLOCK = 8000
_D = 16


def _f32(*shape):
    return jax.ShapeDtypeStruct(shape, jnp.float32)


def _pp_pre_body(p1, w1, b1, w2, b2, out):
    x = jnp.tanh(jnp.dot(p1[...], w1[...],
                         preferred_element_type=jnp.float32) + b1[...])
    out[...] = jnp.tanh(jnp.dot(x, w2[...],
                                preferred_element_type=jnp.float32) + b2[...])


def _pp_pre(p1, w1, b1, w2, b2):
    return pl.pallas_call(
        _pp_pre_body,
        out_shape=_f32(_N_ATOMS, _D),
    )(p1, w1, b1, w2, b2)


def _sc_gather_body(tbl, ind_i, ind_j, out_i, out_j,
                    idx_i, idx_j, rows_i, rows_j, sem_g, sem_w):
    wid = lax.axis_index("s") * _NC + lax.axis_index("c")
    base_w = wid * _PAIRS_PER_W
    gathers = [None] * _N_CHUNKS
    writes = [None] * _N_CHUNKS

    def drain(t):
        b = t % 2
        base = base_w + t * _CHUNK
        gathers[t][0].wait()
        gathers[t][1].wait()
        writes[t] = (
            pltpu.async_copy(rows_i[b], out_i.at[pl.ds(base, _CHUNK)], sem_w),
            pltpu.async_copy(rows_j[b], out_j.at[pl.ds(base, _CHUNK)], sem_w))

    for t in range(_N_CHUNKS):
        b = t % 2
        base = base_w + t * _CHUNK
        if t >= 2:
            writes[t - 2][0].wait()
            writes[t - 2][1].wait()
        pltpu.sync_copy(ind_i.at[pl.ds(base, _CHUNK)], idx_i[b])
        pltpu.sync_copy(ind_j.at[pl.ds(base, _CHUNK)], idx_j[b])
        gathers[t] = (pltpu.async_copy(tbl.at[idx_i[b]], rows_i[b], sem_g),
                      pltpu.async_copy(tbl.at[idx_j[b]], rows_j[b], sem_g))
        if t >= 1:
            drain(t - 1)
    drain(_N_CHUNKS - 1)
    for t in (_N_CHUNKS - 2, _N_CHUNKS - 1):
        writes[t][0].wait()
        writes[t][1].wait()


@functools.cache
def _sc_gather():
    return functools.partial(
        pl.kernel,
        out_type=[_f32(_N_PAIRS, _D), _f32(_N_PAIRS, _D)],
        mesh=plsc.VectorSubcoreMesh(core_axis_name="c", subcore_axis_name="s",
                                    num_cores=_NC, num_subcores=_NS),
        scratch_types=[
            [pltpu.VMEM((_CHUNK,), jnp.int32)] * 2,
            [pltpu.VMEM((_CHUNK,), jnp.int32)] * 2,
            [pltpu.VMEM((_CHUNK, _D), jnp.float32)] * 2,
            [pltpu.VMEM((_CHUNK, _D), jnp.float32)] * 2,
            pltpu.SemaphoreType.DMA,
            pltpu.SemaphoreType.DMA,
        ],
        compiler_params=pltpu.CompilerParams(use_tc_tiling_on_sc=False),
    )(_sc_gather_body)


_PACK_ROWS = _N_PAIRS // 8
_PACK_BLOCK = 800
_GRP = _PACK_BLOCK


def _pair_body(pi, pj, bs_t, wbig, bpi8, tmat, sbig, wii1b, wii2b,
               wii2bt, out_pk, out_t):
    cat = jnp.concatenate([pi[...], pj[...]], axis=1)
    u = jnp.dot(cat, wbig[...], preferred_element_type=jnp.float32) + bpi8[...]
    u = jnp.tanh(u)
    b_all = bs_t[...].T
    tiled = jnp.concatenate(
        [jnp.dot(b_all[_GRP * k:_GRP * (k + 1)], tmat[...],
                 preferred_element_type=jnp.float32) for k in range(8)],
        axis=1)
    w = jnp.dot(u * tiled, sbig[...], preferred_element_type=jnp.float32)
    w = jnp.tanh(jnp.dot(w, wii1b[...], preferred_element_type=jnp.float32))
    y = jnp.tanh(jnp.dot(w, wii2b[...], preferred_element_type=jnp.float32))
    out_pk[...] = y
    yt = jnp.tanh(jnp.dot(wii2bt[...], w.T,
                          preferred_element_type=jnp.float32))
    out_t[...] = jnp.concatenate(
        [yt[16 * k:16 * (k + 1), :] for k in range(8)], axis=1)


def _pair_stage(pi_p, pj_p, basis_t, wbig, bpi8, tmat, sbig, wii1b, wii2b,
                wii2bt):
    nblk = _PACK_ROWS // _PACK_BLOCK
    blk = lambda: pl.BlockSpec((_PACK_BLOCK, 128), lambda i: (i, 0))
    tblk = lambda: pl.BlockSpec((16, 8 * _GRP), lambda i: (0, i))
    full = lambda r, c: pl.BlockSpec((r, c), lambda i: (0, 0))
    return pl.pallas_call(
        _pair_body,
        grid=(nblk,),
        in_specs=[
            blk(), blk(), tblk(),
            full(256, 2048), full(1, 2048),
            full(16, 256), full(2048, 128),
            full(128, 128), full(128, 128), full(128, 128),
        ],
        out_specs=[blk(), tblk()],
        out_shape=[_f32(_PACK_ROWS, 128), _f32(16, _N_PAIRS)],
        compiler_params=pltpu.CompilerParams(fuse_transposed_lhs_in_matmul=True),
    )(pi_p, pj_p, basis_t, wbig, bpi8, tmat, sbig, wii1b, wii2b, wii2bt)


def _sc_scatter_body(ipair, ind_i, zeros, out, idx_v, rows_v, agg, sem):
    cid = lax.axis_index("c")
    sid = lax.axis_index("s")
    wid = sid * _NC + cid
    r0 = sid * _ROWS_PER_TILE
    pltpu.sync_copy(zeros.at[pl.ds(r0, _ROWS_PER_TILE)],
                    agg.at[pl.ds(r0, _ROWS_PER_TILE)])
    plsc.subcore_barrier()
    base_w = wid * _PAIRS_PER_W

    def load(t):
        b = t % 2
        base = base_w + t * _CHUNK
        return (
            pltpu.async_copy(ind_i.at[pl.ds(base, _CHUNK)], idx_v[b], sem),
            pltpu.async_copy(ipair.at[pl.ds(base, _CHUNK)], rows_v[b], sem))

    pend = load(0)
    for t in range(_N_CHUNKS):
        b = t % 2
        pend[0].wait()
        pend[1].wait()
        if t + 1 < _N_CHUNKS:
            pend = load(t + 1)
        pltpu.sync_copy(rows_v[b], agg.at[idx_v[b]], add=True)
    plsc.subcore_barrier()
    pltpu.sync_copy(agg.at[pl.ds(r0, _ROWS_PER_TILE)],
                    out.at[cid, pl.ds(r0, _ROWS_PER_TILE)])


@functools.cache
def _sc_scatter():
    return functools.partial(
        pl.kernel,
        out_type=_f32(_NC, _N_PAD, _D),
        mesh=plsc.VectorSubcoreMesh(core_axis_name="c", subcore_axis_name="s",
                                    num_cores=_NC, num_subcores=_NS),
        scratch_types=[
            [pltpu.VMEM((_CHUNK,), jnp.int32)] * 2,
            [pltpu.VMEM((_CHUNK, _D), jnp.float32)] * 2,
            pltpu.VMEM_SHARED((_N_PAD, _D), jnp.float32),
            pltpu.SemaphoreType.DMA,
        ],
        compiler_params=pltpu.CompilerParams(use_tc_tiling_on_sc=False),
    )(_sc_scatter_body)


def _pp_post_body(a0, a1, w1, w2, out):
    agg = a0[...] + a1[...]
    x = jnp.tanh(jnp.dot(agg, w1[...], preferred_element_type=jnp.float32))
    out[...] = jnp.tanh(jnp.dot(x, w2[...], preferred_element_type=jnp.float32))


def _pp_post(a0, a1, w1, w2):
    return pl.pallas_call(
        _pp_post_body,
        out_shape=_f32(_N_ATOMS, _D),
    )(a0, a1, w1, w2)


def _perm(x):
    nblk = _PACK_ROWS // _PACK_BLOCK
    return x.reshape(nblk, 8, _GRP).transpose(0, 2, 1).reshape(-1)


def kernel(ind_2, p1, basis, W_pre1, b_pre1, W_pre2, b_pre2, W_pi, b_pi,
           W_ii1, W_ii2, W_po1, W_po2):
    ind_i = _perm(ind_2[:, 0])
    ind_j = _perm(ind_2[:, 1])

    p1_in = _pp_pre(p1, W_pre1, b_pre1.reshape(1, -1),
                    W_pre2, b_pre2.reshape(1, -1))

    prop_i, prop_j = _sc_gather()(p1_in, ind_i, ind_j)

    eye16 = jnp.eye(16, dtype=jnp.float32)
    eye8 = jnp.eye(8, dtype=jnp.float32)
    tmat = jnp.tile(eye16, (1, 16))
    smat = jnp.repeat(eye16, 16, axis=0)
    wbig = jnp.concatenate([jnp.kron(eye8, W_pi[:16]),
                            jnp.kron(eye8, W_pi[16:])], axis=0)
    bpi8 = jnp.tile(b_pi, 8).reshape(1, -1)
    sbig = jnp.kron(eye8, smat)
    wii1b = jnp.kron(eye8, W_ii1)
    wii2b = jnp.kron(eye8, W_ii2)
    wii2bt = jnp.kron(eye8, W_ii2.T)
    i_pair_pk, i_pair_t = _pair_stage(
        prop_i.reshape(_PACK_ROWS, 128), prop_j.reshape(_PACK_ROWS, 128),
        basis.T, wbig, bpi8, tmat, sbig, wii1b, wii2b, wii2bt)

    zeros = jnp.zeros((_N_PAD, _D), jnp.float32)
    partials = _sc_scatter()(i_pair_pk.reshape(_N_PAIRS, _D), ind_i, zeros)

    p1_new = _pp_post(partials[0, :_N_ATOMS], partials[1, :_N_ATOMS],
                      W_po1, W_po2)
    return (p1_new, i_pair_t.T)


# speedup vs baseline: 7.9423x; 1.0412x over previous
import functools

import jax
import jax.numpy as jnp
from jax import lax
from jax.experimental import pallas as pl
from jax.experimental.pallas import tpu as pltpu
from jax.experimental.pallas import tpu_sc as plsc

_NC = 2
_NS = 16
_NW = _NC * _NS

_N_ATOMS = 10000
_N_PAIRS = 320000
_N_PAD = 10240
_ROWS_PER_TILE = _N_PAD // _NS
_PAIRS_PER_W = _N_PAIRS // _NW
_CHUNK = 1000
_N_CHUNKS = _PAIRS_PER_W // _CHUNK

_PAIR_BLOCK = 8000
_D = 16


def _f32(*shape):
    return jax.ShapeDtypeStruct(shape, jnp.float32)


def _pp_pre_body(p1, w1, b1, w2, b2, out):
    x = jnp.tanh(jnp.dot(p1[...], w1[...],
                         preferred_element_type=jnp.float32) + b1[...])
    out[...] = jnp.tanh(jnp.dot(x, w2[...],
                                preferred_element_type=jnp.float32) + b2[...])


def _pp_pre(p1, w1, b1, w2, b2):
    return pl.pallas_call(
        _pp_pre_body,
        out_shape=_f32(_N_ATOMS, _D),
    )(p1, w1, b1, w2, b2)


def _sc_gather_body(tbl, ind_i, ind_j, out_i, out_j,
                    idx_i, idx_j, rows_i, rows_j, sem_g, sem_w):
    wid = lax.axis_index("s") * _NC + lax.axis_index("c")
    base_w = wid * _PAIRS_PER_W
    gathers = [None] * _N_CHUNKS
    writes = [None] * _N_CHUNKS

    def drain(t):
        b = t % 2
        base = base_w + t * _CHUNK
        gathers[t][0].wait()
        gathers[t][1].wait()
        writes[t] = (
            pltpu.async_copy(rows_i[b], out_i.at[pl.ds(base, _CHUNK)], sem_w),
            pltpu.async_copy(rows_j[b], out_j.at[pl.ds(base, _CHUNK)], sem_w))

    for t in range(_N_CHUNKS):
        b = t % 2
        base = base_w + t * _CHUNK
        if t >= 2:
            writes[t - 2][0].wait()
            writes[t - 2][1].wait()
        pltpu.sync_copy(ind_i.at[pl.ds(base, _CHUNK)], idx_i[b])
        pltpu.sync_copy(ind_j.at[pl.ds(base, _CHUNK)], idx_j[b])
        gathers[t] = (pltpu.async_copy(tbl.at[idx_i[b]], rows_i[b], sem_g),
                      pltpu.async_copy(tbl.at[idx_j[b]], rows_j[b], sem_g))
        if t >= 1:
            drain(t - 1)
    drain(_N_CHUNKS - 1)
    for t in (_N_CHUNKS - 2, _N_CHUNKS - 1):
        writes[t][0].wait()
        writes[t][1].wait()


@functools.cache
def _sc_gather():
    return functools.partial(
        pl.kernel,
        out_type=[_f32(_N_PAIRS, _D), _f32(_N_PAIRS, _D)],
        mesh=plsc.VectorSubcoreMesh(core_axis_name="c", subcore_axis_name="s",
                                    num_cores=_NC, num_subcores=_NS),
        scratch_types=[
            [pltpu.VMEM((_CHUNK,), jnp.int32)] * 2,
            [pltpu.VMEM((_CHUNK,), jnp.int32)] * 2,
            [pltpu.VMEM((_CHUNK, _D), jnp.float32)] * 2,
            [pltpu.VMEM((_CHUNK, _D), jnp.float32)] * 2,
            pltpu.SemaphoreType.DMA,
            pltpu.SemaphoreType.DMA,
        ],
        compiler_params=pltpu.CompilerParams(use_tc_tiling_on_sc=False),
    )(_sc_gather_body)


_PACK_ROWS = _N_PAIRS // 8
_PACK_BLOCK = 1600
_GRP = _PACK_BLOCK


def _pair_body(pi, pj, bs_t, wbig, bpi8, tmat, sbig, wii1b, wii2b,
               wii2bt, out_pk, out_t):
    cat = jnp.concatenate([pi[...], pj[...]], axis=1)
    u = jnp.dot(cat, wbig[...], preferred_element_type=jnp.float32) + bpi8[...]
    u = jnp.tanh(u)
    b_all = bs_t[...].T
    tiled = jnp.concatenate(
        [jnp.dot(b_all[_GRP * k:_GRP * (k + 1)], tmat[...],
                 preferred_element_type=jnp.float32) for k in range(8)],
        axis=1)
    w = jnp.dot(u * tiled, sbig[...], preferred_element_type=jnp.float32)
    w = jnp.tanh(jnp.dot(w, wii1b[...], preferred_element_type=jnp.float32))
    y = jnp.tanh(jnp.dot(w, wii2b[...], preferred_element_type=jnp.float32))
    out_pk[...] = y
    yt = jnp.tanh(jnp.dot(wii2bt[...], w.T,
                          preferred_element_type=jnp.float32))
    out_t[...] = jnp.concatenate(
        [yt[16 * k:16 * (k + 1), :] for k in range(8)], axis=1)


def _pair_stage(pi_p, pj_p, basis_t, wbig, bpi8, tmat, sbig, wii1b, wii2b,
                wii2bt):
    nblk = _PACK_ROWS // _PACK_BLOCK
    blk = lambda: pl.BlockSpec((_PACK_BLOCK, 128), lambda i: (i, 0))
    tblk = lambda: pl.BlockSpec((16, 8 * _GRP), lambda i: (0, i))
    full = lambda r, c: pl.BlockSpec((r, c), lambda i: (0, 0))
    return pl.pallas_call(
        _pair_body,
        grid=(nblk,),
        in_specs=[
            blk(), blk(), tblk(),
            full(256, 2048), full(1, 2048),
            full(16, 256), full(2048, 128),
            full(128, 128), full(128, 128), full(128, 128),
        ],
        out_specs=[blk(), tblk()],
        out_shape=[_f32(_PACK_ROWS, 128), _f32(16, _N_PAIRS)],
        compiler_params=pltpu.CompilerParams(fuse_transposed_lhs_in_matmul=True),
    )(pi_p, pj_p, basis_t, wbig, bpi8, tmat, sbig, wii1b, wii2b, wii2bt)


def _sc_scatter_body(ipair, ind_i, zeros, out, idx_v, rows_v, agg, sem):
    cid = lax.axis_index("c")
    sid = lax.axis_index("s")
    wid = sid * _NC + cid
    r0 = sid * _ROWS_PER_TILE
    pltpu.sync_copy(zeros.at[pl.ds(r0, _ROWS_PER_TILE)],
                    agg.at[pl.ds(r0, _ROWS_PER_TILE)])
    plsc.subcore_barrier()
    base_w = wid * _PAIRS_PER_W

    def load(t):
        b = t % 2
        base = base_w + t * _CHUNK
        return (
            pltpu.async_copy(ind_i.at[pl.ds(base, _CHUNK)], idx_v[b], sem),
            pltpu.async_copy(ipair.at[pl.ds(base, _CHUNK)], rows_v[b], sem))

    pend = load(0)
    for t in range(_N_CHUNKS):
        b = t % 2
        pend[0].wait()
        pend[1].wait()
        if t + 1 < _N_CHUNKS:
            pend = load(t + 1)
        pltpu.sync_copy(rows_v[b], agg.at[idx_v[b]], add=True)
    plsc.subcore_barrier()
    pltpu.sync_copy(agg.at[pl.ds(r0, _ROWS_PER_TILE)],
                    out.at[cid, pl.ds(r0, _ROWS_PER_TILE)])


@functools.cache
def _sc_scatter():
    return functools.partial(
        pl.kernel,
        out_type=_f32(_NC, _N_PAD, _D),
        mesh=plsc.VectorSubcoreMesh(core_axis_name="c", subcore_axis_name="s",
                                    num_cores=_NC, num_subcores=_NS),
        scratch_types=[
            [pltpu.VMEM((_CHUNK,), jnp.int32)] * 2,
            [pltpu.VMEM((_CHUNK, _D), jnp.float32)] * 2,
            pltpu.VMEM_SHARED((_N_PAD, _D), jnp.float32),
            pltpu.SemaphoreType.DMA,
        ],
        compiler_params=pltpu.CompilerParams(use_tc_tiling_on_sc=False),
    )(_sc_scatter_body)


def _pp_post_body(a0, a1, w1, w2, out):
    agg = a0[...] + a1[...]
    x = jnp.tanh(jnp.dot(agg, w1[...], preferred_element_type=jnp.float32))
    out[...] = jnp.tanh(jnp.dot(x, w2[...], preferred_element_type=jnp.float32))


def _pp_post(a0, a1, w1, w2):
    return pl.pallas_call(
        _pp_post_body,
        out_shape=_f32(_N_ATOMS, _D),
    )(a0, a1, w1, w2)


def _perm(x):
    nblk = _PACK_ROWS // _PACK_BLOCK
    return x.reshape(nblk, 8, _GRP).transpose(0, 2, 1).reshape(-1)


def kernel(ind_2, p1, basis, W_pre1, b_pre1, W_pre2, b_pre2, W_pi, b_pi,
           W_ii1, W_ii2, W_po1, W_po2):
    ind_i = _perm(ind_2[:, 0])
    ind_j = _perm(ind_2[:, 1])

    p1_in = _pp_pre(p1, W_pre1, b_pre1.reshape(1, -1),
                    W_pre2, b_pre2.reshape(1, -1))

    prop_i, prop_j = _sc_gather()(p1_in, ind_i, ind_j)

    eye16 = jnp.eye(16, dtype=jnp.float32)
    eye8 = jnp.eye(8, dtype=jnp.float32)
    tmat = jnp.tile(eye16, (1, 16))
    smat = jnp.repeat(eye16, 16, axis=0)
    wbig = jnp.concatenate([jnp.kron(eye8, W_pi[:16]),
                            jnp.kron(eye8, W_pi[16:])], axis=0)
    bpi8 = jnp.tile(b_pi, 8).reshape(1, -1)
    sbig = jnp.kron(eye8, smat)
    wii1b = jnp.kron(eye8, W_ii1)
    wii2b = jnp.kron(eye8, W_ii2)
    wii2bt = jnp.kron(eye8, W_ii2.T)
    i_pair_pk, i_pair_t = _pair_stage(
        prop_i.reshape(_PACK_ROWS, 128), prop_j.reshape(_PACK_ROWS, 128),
        basis.T, wbig, bpi8, tmat, sbig, wii1b, wii2b, wii2bt)

    zeros = jnp.zeros((_N_PAD, _D), jnp.float32)
    partials = _sc_scatter()(i_pair_pk.reshape(_N_PAIRS, _D), ind_i, zeros)

    p1_new = _pp_post(partials[0, :_N_ATOMS], partials[1, :_N_ATOMS],
                      W_po1, W_po2)
    return (p1_new, i_pair_t.T)
